# Initial kernel scaffold; baseline (speedup 1.0000x reference)
import jax
import jax.numpy as jnp
from jax.experimental import pallas as pl


def kernel(node_feature, edge_index, edge_feature, edge_vec, cat_linear_w, cat_linear_b, l1_fc1_w, l1_fc1_b, l1_fc2_w, l1_fc2_b, l2_fc1_w, l2_fc1_b, l2_fc2_w, l2_fc2_b, node_linear_2_w, node_linear_2_b, bn_gamma, bn_beta, tp1_rexp, tp1_sexp, tp2_gr, tp2_ssum):
    raise NotImplementedError("write your pallas kernel here")


# speedup vs baseline: 1.1375x; 1.1375x over previous
import functools
import math

import numpy as np
import jax
import jax.numpy as jnp
from jax.experimental import pallas as pl
from jax.experimental.pallas import tpu as pltpu

_TE = 512
_TN = 512


def _softplus(x):
    return jnp.where(x > 20.0, x, jnp.log1p(jnp.exp(jnp.minimum(x, 20.0))))


def _round_up(n, m):
    return -(-n // m) * m


def _pad_rows(a, rows):
    if rows == a.shape[0]:
        return a
    return jnp.pad(a, [(0, rows - a.shape[0])] + [(0, 0)] * (a.ndim - 1))


@functools.lru_cache(maxsize=None)
def _consts(ns, nv):
    dv = 3 * nv + 5 * nv
    e2 = np.zeros((2 * nv, dv), np.float32)
    sht = np.zeros((8, dv), np.float32)
    for a in range(nv):
        e2[a, 3 * a:3 * a + 3] = 1.0
        e2[nv + a, 3 * nv + 5 * a:3 * nv + 5 * a + 5] = 1.0
        for k in range(3):
            sht[k, 3 * a + k] = 1.0
        for k in range(5):
            sht[3 + k, 3 * nv + 5 * a + k] = 1.0
    d1 = ns + 8 * nv
    npaths = ns + 2 * nv
    wnum2 = npaths * ns
    c2 = np.zeros((d1, wnum2), np.float32)
    for p in range(ns):
        c2[p, p * ns:(p + 1) * ns] = 1.0
    for a in range(nv):
        p = ns + a
        c2[ns + 3 * a: ns + 3 * a + 3, p * ns:(p + 1) * ns] = 1.0
        p = ns + nv + a
        c2[ns + 3 * nv + 5 * a: ns + 3 * nv + 5 * a + 5, p * ns:(p + 1) * ns] = 1.0
    cbase = 1.0 / math.sqrt(ns + 2 * nv)
    coef2 = np.empty((wnum2,), np.float32)
    coef2[:ns * ns] = cbase
    coef2[ns * ns: ns * ns + nv * ns] = cbase / math.sqrt(3.0)
    coef2[ns * ns + nv * ns:] = cbase / math.sqrt(5.0)
    return e2, sht, c2, coef2


def _sh_parts(v):
    nrm = jnp.sqrt(jnp.sum(v * v, axis=-1, keepdims=True))
    vn = v / jnp.maximum(nrm, 1e-12)
    vx, vy, vz = vn[:, 0:1], vn[:, 1:2], vn[:, 2:3]
    s3, s5, s15 = math.sqrt(3.0), math.sqrt(5.0), math.sqrt(15.0)
    sh1 = s3 * vn
    sh2 = jnp.concatenate(
        [s15 * vx * vz,
         s15 * vx * vy,
         s5 * (vy * vy - 0.5 * (vx * vx + vz * vz)),
         s15 * vy * vz,
         0.5 * s15 * (vz * vz - vx * vx)], axis=-1)
    return jnp.concatenate([sh1, sh2], axis=-1)


def _edge_mlp(ef_ref, w1_ref, b1_ref, w2_ref, b2_ref):
    h = _softplus(jnp.dot(ef_ref[...], w1_ref[...],
                          preferred_element_type=jnp.float32) + b1_ref[...])
    return jnp.dot(h.astype(jnp.bfloat16), w2_ref[...],
                   preferred_element_type=jnp.float32) + b2_ref[...]


def _fold_groups(q, ngroups):
    s = q[:, 0:128]
    for j in range(1, ngroups):
        s = s + q[:, 128 * j:128 * (j + 1)]
    return s


def _fold4(s):
    return s[:, 0:32] + s[:, 32:64] + s[:, 64:96] + s[:, 96:128]


def _fold_to8(s1):
    s1 = s1[:, 0:64] + s1[:, 64:128]
    s1 = s1[:, 0:32] + s1[:, 32:64]
    s1 = s1[:, 0:16] + s1[:, 16:32]
    return s1[:, 0:8] + s1[:, 8:16]


def _tp1_kernel(xg_ref, ev_ref, ef_ref, w1_ref, b1_ref, w2_ref, b2_ref,
                rexp_ref, e2_ref, sht_ref, o_ref, *, ns, nv):
    w = _edge_mlp(ef_ref, w1_ref, b1_ref, w2_ref, b2_ref)
    xfull = jnp.dot(xg_ref[...], rexp_ref[...],
                    preferred_element_type=jnp.float32)
    q = xfull * w
    n0 = ns * ns
    n1 = n0 + ns * nv
    n2 = n1 + ns * nv
    z0 = _fold4(_fold_groups(q[:, 0:n0], n0 // 128))
    v1 = _fold_to8(_fold_groups(q[:, n0:n1], ns * nv // 128))
    v2 = _fold_to8(_fold_groups(q[:, n1:n2], ns * nv // 128))
    vexp = jnp.dot(jnp.concatenate([v1, v2], axis=1).astype(jnp.bfloat16),
                   e2_ref[...], preferred_element_type=jnp.float32)
    shv = jnp.dot(_sh_parts(ev_ref[...]).astype(jnp.bfloat16),
                  sht_ref[...], preferred_element_type=jnp.float32)
    o_ref[...] = jnp.concatenate([z0, vexp * shv], axis=1)


def _tp2_kernel(xg_ref, ev_ref, ef_ref, w1_ref, b1_ref, w2_ref, b2_ref,
                c2_ref, sht_ref, o_ref, *, ns, nv):
    w = _edge_mlp(ef_ref, w1_ref, b1_ref, w2_ref, b2_ref)
    shv = jnp.dot(_sh_parts(ev_ref[...]).astype(jnp.bfloat16),
                  sht_ref[...], preferred_element_type=jnp.float32)
    xg = xg_ref[...].astype(jnp.float32)
    xs = jnp.concatenate([xg[:, 0:ns], xg[:, ns:] * shv], axis=1)
    erep = jnp.dot(xs.astype(jnp.bfloat16), c2_ref[...],
                   preferred_element_type=jnp.float32)
    q = erep * w
    o_ref[...] = _fold4(_fold_groups(q, (ns + 2 * nv) * ns // 128))


def _cat_kernel(x_ref, w_ref, b_ref, o_ref):
    o_ref[...] = jnp.dot(x_ref[...].astype(jnp.bfloat16), w_ref[...],
                         preferred_element_type=jnp.float32) + b_ref[...]


def _head_kernel(x_ref, sc_ref, sh_ref, w_ref, b_ref, skip_ref, o_ref):
    xn = x_ref[...] * sc_ref[...] + sh_ref[...]
    h = _softplus(xn)
    y = _softplus(jnp.dot(h.astype(jnp.bfloat16), w_ref[...],
                          preferred_element_type=jnp.float32) + b_ref[...])
    o_ref[...] = y + skip_ref[...]


def _edge_call(body, xg, ev, ef, mlp, consts, d_out, ns, nv):
    ep, d_in = xg.shape
    ops = (xg, ev, ef) + tuple(mlp) + tuple(consts)
    row_specs = [
        pl.BlockSpec((_TE, d_in), lambda i: (i, 0)),
        pl.BlockSpec((_TE, 3), lambda i: (i, 0)),
        pl.BlockSpec((_TE, ef.shape[1]), lambda i: (i, 0)),
    ]
    const_specs = [pl.BlockSpec(a.shape, lambda i: (0, 0))
                   for a in ops[3:]]
    return pl.pallas_call(
        functools.partial(body, ns=ns, nv=nv),
        out_shape=jax.ShapeDtypeStruct((ep, d_out), jnp.float32),
        grid=(ep // _TE,),
        in_specs=row_specs + const_specs,
        out_specs=pl.BlockSpec((_TE, d_out), lambda i: (i, 0)),
        compiler_params=pltpu.CompilerParams(dimension_semantics=("parallel",)),
    )(*ops)


def kernel(node_feature, edge_index, edge_feature, edge_vec,
           cat_linear_w, cat_linear_b, l1_fc1_w, l1_fc1_b,
           l1_fc2_w, l1_fc2_b, l2_fc1_w, l2_fc1_b, l2_fc2_w, l2_fc2_b,
           node_linear_2_w, node_linear_2_b, bn_gamma, bn_beta,
           tp1_rexp, tp1_sexp, tp2_gr, tp2_ssum):
    f32, bf16 = jnp.float32, jnp.bfloat16
    n = node_feature.shape[0]
    e = edge_vec.shape[0]
    ns = bn_gamma.shape[0]
    d1 = tp1_sexp.shape[1]
    nv = (d1 - ns) // 8
    out_ch = node_linear_2_w.shape[1]
    edge_src, edge_dst = edge_index[0], edge_index[1]

    e2_np, sht_np, c2_np, coef2_np = _consts(ns, nv)
    e2 = jnp.asarray(e2_np, bf16)
    sht = jnp.asarray(sht_np, bf16)
    c2 = jnp.asarray(c2_np, bf16)
    coef2 = jnp.asarray(coef2_np, f32)
    coef1 = 1.0 / math.sqrt(ns)

    ep = _round_up(e, _TE)
    ev_p = _pad_rows(edge_vec, ep)
    ef_p = _pad_rows(edge_feature.astype(bf16), ep)
    dst_p = jnp.pad(edge_dst, (0, ep - e))

    npad = _round_up(n, _TN)
    ncat = pl.pallas_call(
        _cat_kernel,
        out_shape=jax.ShapeDtypeStruct((npad, cat_linear_w.shape[1]), f32),
        grid=(npad // _TN,),
        in_specs=[
            pl.BlockSpec((_TN, node_feature.shape[1]), lambda i: (i, 0)),
            pl.BlockSpec(cat_linear_w.shape, lambda i: (0, 0)),
            pl.BlockSpec((1, cat_linear_w.shape[1]), lambda i: (0, 0)),
        ],
        out_specs=pl.BlockSpec((_TN, cat_linear_w.shape[1]), lambda i: (i, 0)),
        compiler_params=pltpu.CompilerParams(dimension_semantics=("parallel",)),
    )(_pad_rows(node_feature, npad), cat_linear_w.astype(bf16),
      cat_linear_b.reshape(1, -1))[:n]
    h = ncat[:, :ns]
    skip = ncat[:, ns:]

    cnt = jnp.zeros((n,), f32).at[edge_src].add(1.0)
    inv_cnt = (1.0 / jnp.maximum(cnt, 1.0))[:, None]

    xg1 = jnp.take(h.astype(bf16), dst_p, axis=0)
    w11 = l1_fc1_w.astype(bf16)
    b11 = l1_fc1_b.reshape(1, -1)
    w12 = (l1_fc2_w * coef1).astype(bf16)
    b12 = (l1_fc2_b * coef1).reshape(1, -1)
    tp1 = _edge_call(_tp1_kernel, xg1, ev_p, ef_p, (w11, b11, w12, b12),
                     (tp1_rexp.astype(bf16), e2, sht), d1, ns, nv)
    h1 = jnp.zeros((n, d1), f32).at[edge_src].add(tp1[:e]) * inv_cnt
    h1 = h1 + jnp.pad(h, ((0, 0), (0, d1 - ns)))

    xg2 = jnp.take(h1.astype(bf16), dst_p, axis=0)
    w21 = l2_fc1_w.astype(bf16)
    b21 = l2_fc1_b.reshape(1, -1)
    w22 = (l2_fc2_w * coef2[None, :]).astype(bf16)
    b22 = (l2_fc2_b * coef2).reshape(1, -1)
    tp2 = _edge_call(_tp2_kernel, xg2, ev_p, ef_p, (w21, b21, w22, b22),
                     (c2, sht), ns, ns, nv)
    h2 = jnp.zeros((n, ns), f32).at[edge_src].add(tp2[:e]) * inv_cnt

    mean = jnp.mean(h2, axis=0, keepdims=True)
    var = jnp.mean((h2 - mean) ** 2, axis=0, keepdims=True)
    inv = jax.lax.rsqrt(var + 1e-5)
    scale = bn_gamma.reshape(1, -1) * inv
    shift = bn_beta.reshape(1, -1) - mean * scale
    out = pl.pallas_call(
        _head_kernel,
        out_shape=jax.ShapeDtypeStruct((npad, out_ch), f32),
        grid=(npad // _TN,),
        in_specs=[
            pl.BlockSpec((_TN, ns), lambda i: (i, 0)),
            pl.BlockSpec((1, ns), lambda i: (0, 0)),
            pl.BlockSpec((1, ns), lambda i: (0, 0)),
            pl.BlockSpec((ns, out_ch), lambda i: (0, 0)),
            pl.BlockSpec((1, out_ch), lambda i: (0, 0)),
            pl.BlockSpec((_TN, out_ch), lambda i: (i, 0)),
        ],
        out_specs=pl.BlockSpec((_TN, out_ch), lambda i: (i, 0)),
        compiler_params=pltpu.CompilerParams(dimension_semantics=("parallel",)),
    )(_pad_rows(h2, npad), scale, shift, node_linear_2_w.astype(bf16),
      node_linear_2_b.reshape(1, -1), _pad_rows(skip, npad))
    return out[:n]


# speedup vs baseline: 1.5921x; 1.3997x over previous
import functools
import math

import numpy as np
import jax
import jax.numpy as jnp
from jax.experimental import pallas as pl
from jax.experimental.pallas import tpu as pltpu

_TE = 512
_TN = 512


def _softplus(x):
    return jnp.where(x > 20.0, x, jnp.log1p(jnp.exp(jnp.minimum(x, 20.0))))


def _round_up(n, m):
    return -(-n // m) * m


def _pad_rows(a, rows):
    if rows == a.shape[0]:
        return a
    return jnp.pad(a, [(0, rows - a.shape[0])] + [(0, 0)] * (a.ndim - 1))


@functools.lru_cache(maxsize=None)
def _consts(ns, nv):
    dv = 3 * nv + 5 * nv
    e2 = np.zeros((2 * nv, dv), np.float32)
    sht = np.zeros((8, dv), np.float32)
    for a in range(nv):
        e2[a, 3 * a:3 * a + 3] = 1.0
        e2[nv + a, 3 * nv + 5 * a:3 * nv + 5 * a + 5] = 1.0
        for k in range(3):
            sht[k, 3 * a + k] = 1.0
        for k in range(5):
            sht[3 + k, 3 * nv + 5 * a + k] = 1.0
    d1 = ns + 8 * nv
    npaths = ns + 2 * nv
    wnum2 = npaths * ns
    c2 = np.zeros((d1, wnum2), np.float32)
    for p in range(ns):
        c2[p, p * ns:(p + 1) * ns] = 1.0
    for a in range(nv):
        p = ns + a
        c2[ns + 3 * a: ns + 3 * a + 3, p * ns:(p + 1) * ns] = 1.0
        p = ns + nv + a
        c2[ns + 3 * nv + 5 * a: ns + 3 * nv + 5 * a + 5, p * ns:(p + 1) * ns] = 1.0
    cbase = 1.0 / math.sqrt(ns + 2 * nv)
    coef2 = np.empty((wnum2,), np.float32)
    coef2[:ns * ns] = cbase
    coef2[ns * ns: ns * ns + nv * ns] = cbase / math.sqrt(3.0)
    coef2[ns * ns + nv * ns:] = cbase / math.sqrt(5.0)
    return e2, sht, c2, coef2


def _sh_parts(v):
    nrm = jnp.sqrt(jnp.sum(v * v, axis=-1, keepdims=True))
    vn = v / jnp.maximum(nrm, 1e-12)
    vx, vy, vz = vn[:, 0:1], vn[:, 1:2], vn[:, 2:3]
    s3, s5, s15 = math.sqrt(3.0), math.sqrt(5.0), math.sqrt(15.0)
    sh1 = s3 * vn
    sh2 = jnp.concatenate(
        [s15 * vx * vz,
         s15 * vx * vy,
         s5 * (vy * vy - 0.5 * (vx * vx + vz * vz)),
         s15 * vy * vz,
         0.5 * s15 * (vz * vz - vx * vx)], axis=-1)
    return jnp.concatenate([sh1, sh2], axis=-1)


def _edge_mlp(ef_ref, w1_ref, b1_ref, w2_ref, b2_ref):
    h = _softplus(jnp.dot(ef_ref[...], w1_ref[...],
                          preferred_element_type=jnp.float32) + b1_ref[...])
    return jnp.dot(h.astype(jnp.bfloat16), w2_ref[...],
                   preferred_element_type=jnp.float32) + b2_ref[...]


def _fold_groups(q, ngroups):
    s = q[:, 0:128]
    for j in range(1, ngroups):
        s = s + q[:, 128 * j:128 * (j + 1)]
    return s


def _fold4(s):
    return s[:, 0:32] + s[:, 32:64] + s[:, 64:96] + s[:, 96:128]


def _fold_to8(s1):
    s1 = s1[:, 0:64] + s1[:, 64:128]
    s1 = s1[:, 0:32] + s1[:, 32:64]
    s1 = s1[:, 0:16] + s1[:, 16:32]
    return s1[:, 0:8] + s1[:, 8:16]


def _gather_rows(idx_ref, tab_ref, xg_ref):
    for k in range(_TE):
        xg_ref[pl.ds(k, 1), :] = tab_ref[pl.ds(idx_ref[0, 0, k], 1), :]


def _tp1_kernel(idx_ref, ev_ref, ef_ref, tab_ref, w1_ref, b1_ref, w2_ref,
                b2_ref, rexp_ref, e2_ref, sht_ref, o_ref, xg_ref, *, ns, nv):
    w = _edge_mlp(ef_ref, w1_ref, b1_ref, w2_ref, b2_ref)
    _gather_rows(idx_ref, tab_ref, xg_ref)
    xfull = jnp.dot(xg_ref[...].astype(jnp.bfloat16), rexp_ref[...],
                    preferred_element_type=jnp.float32)
    q = xfull * w
    n0 = ns * ns
    n1 = n0 + ns * nv
    n2 = n1 + ns * nv
    z0 = _fold4(_fold_groups(q[:, 0:n0], n0 // 128))
    v1 = _fold_to8(_fold_groups(q[:, n0:n1], ns * nv // 128))
    v2 = _fold_to8(_fold_groups(q[:, n1:n2], ns * nv // 128))
    vexp = jnp.dot(jnp.concatenate([v1, v2], axis=1).astype(jnp.bfloat16),
                   e2_ref[...], preferred_element_type=jnp.float32)
    shv = jnp.dot(_sh_parts(ev_ref[...]).astype(jnp.bfloat16),
                  sht_ref[...], preferred_element_type=jnp.float32)
    o_ref[...] = jnp.concatenate([z0, vexp * shv], axis=1)


def _tp2_kernel(idx_ref, ev_ref, ef_ref, tab_ref, w1_ref, b1_ref, w2_ref,
                b2_ref, c2_ref, sht_ref, o_ref, xg_ref, *, ns, nv):
    w = _edge_mlp(ef_ref, w1_ref, b1_ref, w2_ref, b2_ref)
    _gather_rows(idx_ref, tab_ref, xg_ref)
    shv = jnp.dot(_sh_parts(ev_ref[...]).astype(jnp.bfloat16),
                  sht_ref[...], preferred_element_type=jnp.float32)
    xg = xg_ref[...]
    xs = jnp.concatenate([xg[:, 0:ns], xg[:, ns:] * shv], axis=1)
    erep = jnp.dot(xs.astype(jnp.bfloat16), c2_ref[...],
                   preferred_element_type=jnp.float32)
    q = erep * w
    o_ref[...] = _fold4(_fold_groups(q, (ns + 2 * nv) * ns // 128))


def _cat_kernel(x_ref, w_ref, b_ref, o_ref):
    o_ref[...] = jnp.dot(x_ref[...].astype(jnp.bfloat16), w_ref[...],
                         preferred_element_type=jnp.float32) + b_ref[...]


def _head_kernel(x_ref, sc_ref, sh_ref, w_ref, b_ref, skip_ref, o_ref):
    xn = x_ref[...] * sc_ref[...] + sh_ref[...]
    h = _softplus(xn)
    y = _softplus(jnp.dot(h.astype(jnp.bfloat16), w_ref[...],
                          preferred_element_type=jnp.float32) + b_ref[...])
    o_ref[...] = y + skip_ref[...]


def _edge_call(body, idx2d, ev, ef, tab, mlp, consts, d_out, ns, nv):
    ep = ev.shape[0]
    ops = (idx2d, ev, ef, tab) + tuple(mlp) + tuple(consts)
    row_specs = [
        pl.BlockSpec((1, 1, _TE), lambda i: (i, 0, 0), memory_space=pltpu.SMEM),
        pl.BlockSpec((_TE, 3), lambda i: (i, 0)),
        pl.BlockSpec((_TE, ef.shape[1]), lambda i: (i, 0)),
    ]
    const_specs = [pl.BlockSpec(a.shape, lambda i: (0, 0))
                   for a in ops[3:]]
    return pl.pallas_call(
        functools.partial(body, ns=ns, nv=nv),
        out_shape=jax.ShapeDtypeStruct((ep, d_out), jnp.float32),
        grid=(ep // _TE,),
        in_specs=row_specs + const_specs,
        out_specs=pl.BlockSpec((_TE, d_out), lambda i: (i, 0)),
        scratch_shapes=[pltpu.VMEM((_TE, tab.shape[1]), jnp.float32)],
        compiler_params=pltpu.CompilerParams(dimension_semantics=("parallel",)),
    )(*ops)


def kernel(node_feature, edge_index, edge_feature, edge_vec,
           cat_linear_w, cat_linear_b, l1_fc1_w, l1_fc1_b,
           l1_fc2_w, l1_fc2_b, l2_fc1_w, l2_fc1_b, l2_fc2_w, l2_fc2_b,
           node_linear_2_w, node_linear_2_b, bn_gamma, bn_beta,
           tp1_rexp, tp1_sexp, tp2_gr, tp2_ssum):
    f32, bf16 = jnp.float32, jnp.bfloat16
    n = node_feature.shape[0]
    e = edge_vec.shape[0]
    ns = bn_gamma.shape[0]
    d1 = tp1_sexp.shape[1]
    nv = (d1 - ns) // 8
    out_ch = node_linear_2_w.shape[1]
    edge_src, edge_dst = edge_index[0], edge_index[1]

    e2_np, sht_np, c2_np, coef2_np = _consts(ns, nv)
    e2 = jnp.asarray(e2_np, bf16)
    sht = jnp.asarray(sht_np, bf16)
    c2 = jnp.asarray(c2_np, bf16)
    coef2 = jnp.asarray(coef2_np, f32)
    coef1 = 1.0 / math.sqrt(ns)

    ep = _round_up(e, _TE)
    ev_p = _pad_rows(edge_vec, ep)
    ef_p = _pad_rows(edge_feature.astype(bf16), ep)
    dst_p = jnp.pad(edge_dst, (0, ep - e))

    npad = _round_up(n, _TN)
    ncat = pl.pallas_call(
        _cat_kernel,
        out_shape=jax.ShapeDtypeStruct((npad, cat_linear_w.shape[1]), f32),
        grid=(npad // _TN,),
        in_specs=[
            pl.BlockSpec((_TN, node_feature.shape[1]), lambda i: (i, 0)),
            pl.BlockSpec(cat_linear_w.shape, lambda i: (0, 0)),
            pl.BlockSpec((1, cat_linear_w.shape[1]), lambda i: (0, 0)),
        ],
        out_specs=pl.BlockSpec((_TN, cat_linear_w.shape[1]), lambda i: (i, 0)),
        compiler_params=pltpu.CompilerParams(dimension_semantics=("parallel",)),
    )(_pad_rows(node_feature, npad), cat_linear_w.astype(bf16),
      cat_linear_b.reshape(1, -1))[:n]
    h = ncat[:, :ns]
    skip = ncat[:, ns:]

    cnt = jnp.zeros((n,), f32).at[edge_src].add(1.0)
    inv_cnt = (1.0 / jnp.maximum(cnt, 1.0))[:, None]

    dst2d = dst_p.reshape(ep // _TE, 1, _TE)
    w11 = l1_fc1_w.astype(bf16)
    b11 = l1_fc1_b.reshape(1, -1)
    w12 = (l1_fc2_w * coef1).astype(bf16)
    b12 = (l1_fc2_b * coef1).reshape(1, -1)
    tp1 = _edge_call(_tp1_kernel, dst2d, ev_p, ef_p, h,
                     (w11, b11, w12, b12),
                     (tp1_rexp.astype(bf16), e2, sht), d1, ns, nv)
    h1 = jnp.zeros((n, d1), f32).at[edge_src].add(tp1[:e]) * inv_cnt
    h1 = h1 + jnp.pad(h, ((0, 0), (0, d1 - ns)))

    w21 = l2_fc1_w.astype(bf16)
    b21 = l2_fc1_b.reshape(1, -1)
    w22 = (l2_fc2_w * coef2[None, :]).astype(bf16)
    b22 = (l2_fc2_b * coef2).reshape(1, -1)
    tp2 = _edge_call(_tp2_kernel, dst2d, ev_p, ef_p, h1,
                     (w21, b21, w22, b22),
                     (c2, sht), ns, ns, nv)
    h2 = jnp.zeros((n, ns), f32).at[edge_src].add(tp2[:e]) * inv_cnt

    mean = jnp.mean(h2, axis=0, keepdims=True)
    var = jnp.mean((h2 - mean) ** 2, axis=0, keepdims=True)
    inv = jax.lax.rsqrt(var + 1e-5)
    scale = bn_gamma.reshape(1, -1) * inv
    shift = bn_beta.reshape(1, -1) - mean * scale
    out = pl.pallas_call(
        _head_kernel,
        out_shape=jax.ShapeDtypeStruct((npad, out_ch), f32),
        grid=(npad // _TN,),
        in_specs=[
            pl.BlockSpec((_TN, ns), lambda i: (i, 0)),
            pl.BlockSpec((1, ns), lambda i: (0, 0)),
            pl.BlockSpec((1, ns), lambda i: (0, 0)),
            pl.BlockSpec((ns, out_ch), lambda i: (0, 0)),
            pl.BlockSpec((1, out_ch), lambda i: (0, 0)),
            pl.BlockSpec((_TN, out_ch), lambda i: (i, 0)),
        ],
        out_specs=pl.BlockSpec((_TN, out_ch), lambda i: (i, 0)),
        compiler_params=pltpu.CompilerParams(dimension_semantics=("parallel",)),
    )(_pad_rows(h2, npad), scale, shift, node_linear_2_w.astype(bf16),
      node_linear_2_b.reshape(1, -1), _pad_rows(skip, npad))
    return out[:n]
